# baseline (speedup 1.0000x reference)
import jax
import jax.numpy as jnp
import numpy as np
from jax.experimental import pallas as pl

B = 32768
E = 8
D = 768
OUT = 64
TB = 4096

_P = np.zeros((E, E * OUT), dtype=np.float32)
for _e in range(E):
    _P[_e, _e * OUT:(_e + 1) * OUT] = 1.0
_S = np.zeros((E * OUT, OUT), dtype=np.float32)
for _e in range(E):
    _S[_e * OUT:(_e + 1) * OUT, :] = np.eye(OUT, dtype=np.float32)


def _moe_block_kernel(gates_ref, xa_ref, xb_ref, w2_ref, b2_ref, p_ref,
                      s_ref, out_ref):
    g = gates_ref[...]
    w2 = w2_ref[...]
    H = TB // 2
    gexp = jnp.dot(g, p_ref[...], preferred_element_type=jnp.float32)
    eps = jnp.float32(np.finfo(np.float64).eps)
    for h, xref in ((0, xa_ref), (1, xb_ref)):
        y = jnp.dot(xref[...], w2, preferred_element_type=jnp.float32)
        y = y + b2_ref[...]
        z = jnp.exp(y)
        acc = jnp.dot(z * gexp[h * H:(h + 1) * H, :], s_ref[...],
                      preferred_element_type=jnp.float32)
        acc = jnp.where(acc == 0, eps, acc)
        out_ref[h * H:(h + 1) * H, :] = jnp.log(acc)


@jax.jit
def kernel(gates, inp, W, b):
    w2 = W.transpose(1, 0, 2).reshape(D, E * OUT)
    b2 = b.reshape(1, E * OUT)
    p = jnp.asarray(_P)
    s = jnp.asarray(_S)
    grid = (B // TB,)
    H = TB // 2
    return pl.pallas_call(
        _moe_block_kernel,
        grid=grid,
        in_specs=[
            pl.BlockSpec((TB, E), lambda i: (i, 0)),
            pl.BlockSpec((H, D), lambda i: (2 * i, 0)),
            pl.BlockSpec((H, D), lambda i: (2 * i + 1, 0)),
            pl.BlockSpec((D, E * OUT), lambda i: (0, 0)),
            pl.BlockSpec((1, E * OUT), lambda i: (0, 0)),
            pl.BlockSpec((E, E * OUT), lambda i: (0, 0)),
            pl.BlockSpec((E * OUT, OUT), lambda i: (0, 0)),
        ],
        out_specs=pl.BlockSpec((TB, OUT), lambda i: (i, 0)),
        out_shape=jax.ShapeDtypeStruct((B, OUT), jnp.float32),
    )(gates, inp, inp, w2, b2, p, s)


# speedup vs baseline: 1.0413x; 1.0413x over previous
import jax
import jax.numpy as jnp
import numpy as np
from jax.experimental import pallas as pl

B = 32768
E = 8
D = 768
OUT = 64
TB = 4096

_P = np.zeros((E, E * OUT), dtype=np.float32)
for _e in range(E):
    _P[_e, _e * OUT:(_e + 1) * OUT] = 1.0
_S = np.zeros((E * OUT, OUT), dtype=np.float32)
for _e in range(E):
    _S[_e * OUT:(_e + 1) * OUT, :] = np.eye(OUT, dtype=np.float32)


def _moe_block_kernel(gates_ref, inp_ref, w2_ref, b2_ref, p_ref, s_ref,
                      out_ref):
    x = inp_ref[...]
    g = gates_ref[...]
    y = jnp.dot(x, w2_ref[...], preferred_element_type=jnp.float32)
    y = y + b2_ref[...]
    z = jnp.exp(y)
    gexp = jnp.dot(g, p_ref[...], preferred_element_type=jnp.float32)
    acc = jnp.dot(z * gexp, s_ref[...], preferred_element_type=jnp.float32)
    eps = jnp.float32(np.finfo(np.float64).eps)
    acc = jnp.where(acc == 0, eps, acc)
    out_ref[...] = jnp.log(acc)


@jax.jit
def kernel(gates, inp, W, b):
    w2 = W.transpose(1, 0, 2).reshape(D, E * OUT)
    b2 = b.reshape(1, E * OUT)
    p = jnp.asarray(_P)
    s = jnp.asarray(_S)
    grid = (B // TB,)
    return pl.pallas_call(
        _moe_block_kernel,
        grid=grid,
        in_specs=[
            pl.BlockSpec((TB, E), lambda i: (i, 0)),
            pl.BlockSpec((TB, D), lambda i: (i, 0)),
            pl.BlockSpec((D, E * OUT), lambda i: (0, 0)),
            pl.BlockSpec((1, E * OUT), lambda i: (0, 0)),
            pl.BlockSpec((E, E * OUT), lambda i: (0, 0)),
            pl.BlockSpec((E * OUT, OUT), lambda i: (0, 0)),
        ],
        out_specs=pl.BlockSpec((TB, OUT), lambda i: (i, 0)),
        out_shape=jax.ShapeDtypeStruct((B, OUT), jnp.float32),
    )(gates, inp, w2, b2, p, s)
